# baseline (speedup 1.0000x reference)
import functools

import jax
import jax.numpy as jnp
from jax import lax
from jax.experimental import pallas as pl
from jax.experimental.pallas import tpu as pltpu
from jax.experimental.pallas import tpu_sc as plsc

N = 10000
E = 320000
D = 128

NC = 2
NS = 16
NW = NC * NS

EW = E // NW
CH = 96
NP = 4
NCHH = 27
EWP = NP * NCHH * CH
NA = N + 8

RB = 80
NB = N // RB
KMAX = -(-NB // NS)


def _sc_aggregate(x, src_arr, dst_arr):
    mesh = plsc.VectorSubcoreMesh(core_axis_name="c", subcore_axis_name="s")

    @functools.partial(
        pl.kernel,
        out_type=jax.ShapeDtypeStruct((NC, N, D), jnp.float32),
        mesh=mesh,
        scratch_types=[
            pltpu.VMEM((NCHH, CH), jnp.int32),
            pltpu.VMEM((NCHH, CH), jnp.int32),
            pltpu.VMEM((CH, D), jnp.float32),
            pltpu.VMEM((CH, D), jnp.float32),
            pltpu.VMEM_SHARED((NA, D), jnp.float32),
            pltpu.SemaphoreType.DMA,
            pltpu.SemaphoreType.DMA,
            pltpu.SemaphoreType.DMA,
            pltpu.SemaphoreType.DMA,
        ],
    )
    def agg_kernel(x_hbm, src_hbm, dst_hbm, out_hbm, src_v, dst_v, rows0,
                   rows1, accum_sh, gsem0, gsem1, ssem0, ssem1):
        bounce_v = rows0.at[pl.ds(0, RB)]
        c = lax.axis_index("c")
        s = lax.axis_index("s")
        w = c * NS + s
        for k in range(KMAX):
            j = s + NS * k

            @pl.when(j < NB)
            def _():
                r0 = j * RB
                pltpu.sync_copy(x_hbm.at[pl.ds(r0, RB)], bounce_v)
                pltpu.sync_copy(bounce_v, accum_sh.at[pl.ds(r0, RB)])

        plsc.subcore_barrier()

        def gather(a, buf, sem):
            pltpu.async_copy(x_hbm.at[src_v.at[a]], buf, sem)

        def gwait(buf, sem):
            pltpu.make_async_copy(x_hbm.at[src_v.at[0]], buf, sem).wait()

        def scat(a, buf, sem):
            pltpu.async_copy(buf, accum_sh.at[dst_v.at[a]], sem, add=True)

        def swait(buf, sem):
            pltpu.make_async_copy(buf, accum_sh.at[dst_v.at[0]], sem).wait()

        NT = (NCHH + 1) // 2

        def body(t2, carry):
            a = 2 * t2

            @pl.when(t2 > 0)
            def _():
                swait(rows1, ssem1)

            @pl.when(a + 1 < NCHH)
            def _():
                gather(a + 1, rows1, gsem1)

            gwait(rows0, gsem0)
            scat(a, rows0, ssem0)
            swait(rows0, ssem0)

            @pl.when(a + 2 < NCHH)
            def _():
                gather(a + 2, rows0, gsem0)

            @pl.when(a + 1 < NCHH)
            def _():
                gwait(rows1, gsem1)
                scat(a + 1, rows1, ssem1)

            return carry

        for qp in range(NP):
            pltpu.sync_copy(src_hbm.at[w, qp], src_v)
            pltpu.sync_copy(dst_hbm.at[w, qp], dst_v)
            gather(0, rows0, gsem0)
            lax.fori_loop(0, NT, body, 0)
        plsc.subcore_barrier()
        for k in range(KMAX):
            j = s + NS * k

            @pl.when(j < NB)
            def _():
                r0 = j * RB
                pltpu.sync_copy(accum_sh.at[pl.ds(r0, RB)], bounce_v)
                pltpu.sync_copy(bounce_v, out_hbm.at[c, pl.ds(r0, RB)])

    return agg_kernel(x, src_arr, dst_arr)


def _layer0_body(a_ref, x_ref, w1_ref, b1_ref, w2_ref, b2_ref, g_ref, be_ref,
                 al_ref, lw_ref, lb_ref, h_out_ref, head_ref):
    z = a_ref[0] + a_ref[1] - x_ref[...]
    t = jnp.maximum(jnp.dot(z, w1_ref[...], preferred_element_type=jnp.float32)
                    + b1_ref[...], 0.0)
    u = jnp.dot(t, w2_ref[...], preferred_element_type=jnp.float32) + b2_ref[...]
    m = jnp.mean(u, axis=0, keepdims=True)
    v = jnp.mean((u - m) ** 2, axis=0, keepdims=True)
    bn = (u - m) / jnp.sqrt(v + 1e-5) * g_ref[...] + be_ref[...]
    h_out_ref[...] = jnp.where(bn > 0, bn, al_ref[...] * bn)
    pooled = jnp.max(x_ref[...], axis=0, keepdims=True)
    head_ref[...] = (jnp.dot(pooled, lw_ref[...], preferred_element_type=jnp.float32)
                     + lb_ref[...])


def _layer1_body(a_ref, x_ref, w1_ref, b1_ref, w2_ref, b2_ref, g_ref, be_ref,
                 al_ref, lwx_ref, lbx_ref, lwh_ref, lbh_ref,
                 headx_ref, headh_ref):
    z = a_ref[0] + a_ref[1] - x_ref[...]
    t = jnp.maximum(jnp.dot(z, w1_ref[...], preferred_element_type=jnp.float32)
                    + b1_ref[...], 0.0)
    u = jnp.dot(t, w2_ref[...], preferred_element_type=jnp.float32) + b2_ref[...]
    m = jnp.mean(u, axis=0, keepdims=True)
    v = jnp.mean((u - m) ** 2, axis=0, keepdims=True)
    bn = (u - m) / jnp.sqrt(v + 1e-5) * g_ref[...] + be_ref[...]
    hn = jnp.where(bn > 0, bn, al_ref[...] * bn)
    pooledx = jnp.max(x_ref[...], axis=0, keepdims=True)
    headx_ref[...] = (jnp.dot(pooledx, lwx_ref[...],
                              preferred_element_type=jnp.float32) + lbx_ref[...])
    pooledh = jnp.max(hn, axis=0, keepdims=True)
    headh_ref[...] = (jnp.dot(pooledh, lwh_ref[...],
                              preferred_element_type=jnp.float32) + lbh_ref[...])


def kernel(h, edge_index, W1_0, b1_0, W2_0, b2_0, gamma_0, beta_0,
           W1_1, b1_1, W2_1, b2_1, gamma_1, beta_1, prelu_a,
           LW0, Lb0, LW1, Lb1, LW2, Lb2):
    alpha = jnp.broadcast_to(prelu_a, (1, D)).astype(jnp.float32)
    r = lambda v: jnp.reshape(v, (1, D))
    pad = ((0, 0), (0, EWP - EW))
    src_arr = jnp.pad(edge_index[0].reshape(NW, EW), pad,
                      constant_values=0).reshape(NW, NP, NCHH, CH)
    dst_arr = jnp.pad(edge_index[1].reshape(NW, EW), pad,
                      constant_values=N).reshape(NW, NP, NCHH, CH)

    a = _sc_aggregate(h, src_arr, dst_arr)
    h1, head0 = pl.pallas_call(
        _layer0_body,
        out_shape=[jax.ShapeDtypeStruct((N, D), jnp.float32),
                   jax.ShapeDtypeStruct((1, D), jnp.float32)],
    )(a, h, W1_0, r(b1_0), W2_0, r(b2_0), r(gamma_0), r(beta_0), alpha,
      LW0, r(Lb0))

    b = _sc_aggregate(h1, src_arr, dst_arr)
    head1, head2 = pl.pallas_call(
        _layer1_body,
        out_shape=[jax.ShapeDtypeStruct((1, D), jnp.float32),
                   jax.ShapeDtypeStruct((1, D), jnp.float32)],
    )(b, h1, W1_1, r(b1_1), W2_1, r(b2_1), r(gamma_1), r(beta_1), alpha,
      LW1, r(Lb1), LW2, r(Lb2))

    stacked = jnp.stack([head0, head1, head2], axis=-1)
    return stacked.reshape(1, -1)


# speedup vs baseline: 3.8416x; 3.8416x over previous
import functools

import jax
import jax.numpy as jnp
from jax import lax
from jax.experimental import pallas as pl
from jax.experimental.pallas import tpu as pltpu
from jax.experimental.pallas import tpu_sc as plsc

N = 10000
E = 320000
D = 128

NC = 2
NS = 16
NW = NC * NS

EW = E // NW
CH = 80
NP = 5
NCHH = 25
NA = N

RB = 80
NB = N // RB
KMAX = -(-NB // NS)


def _sc_aggregate(x, src_arr, dst_arr):
    mesh = plsc.VectorSubcoreMesh(core_axis_name="c", subcore_axis_name="s")

    @functools.partial(
        pl.kernel,
        out_type=jax.ShapeDtypeStruct((NC, N, D), jnp.float32),
        mesh=mesh,
        scratch_types=[
            pltpu.VMEM((NCHH, CH), jnp.int32),
            pltpu.VMEM((NCHH, CH), jnp.int32),
            pltpu.VMEM((CH, D), jnp.float32),
            pltpu.VMEM((CH, D), jnp.float32),
            pltpu.VMEM_SHARED((NA, D), jnp.float32),
            pltpu.SemaphoreType.DMA,
            pltpu.SemaphoreType.DMA,
            pltpu.SemaphoreType.DMA,
            pltpu.SemaphoreType.DMA,
        ],
    )
    def agg_kernel(x_hbm, src_hbm, dst_hbm, out_hbm, src_v, dst_v, rows0,
                   rows1, accum_sh, gsem0, gsem1, ssem0, ssem1):
        c = lax.axis_index("c")
        s = lax.axis_index("s")
        w = c * NS + s
        pltpu.sync_copy(src_hbm.at[w, 0], src_v)
        pltpu.sync_copy(dst_hbm.at[w, 0], dst_v)
        pltpu.async_copy(x_hbm.at[src_v.at[0]], rows0, gsem0)
        zv = jnp.zeros((16,), jnp.float32)

        def zbody(i, carry):
            rows1[i // 8, pl.ds((i % 8) * 16, 16)] = zv
            return carry

        lax.fori_loop(0, RB * 8, zbody, 0)
        for k in range(KMAX):
            j = s + NS * k

            @pl.when(j < NB)
            def _():
                pltpu.async_copy(rows1, accum_sh.at[pl.ds(j * RB, RB)], ssem0)

        for k in range(KMAX):
            j = s + NS * k

            @pl.when(j < NB)
            def _():
                pltpu.make_async_copy(rows1,
                                      accum_sh.at[pl.ds(0, RB)], ssem0).wait()

        plsc.subcore_barrier()

        def gather(a, buf, sem):
            pltpu.async_copy(x_hbm.at[src_v.at[a]], buf, sem)

        def gwait(buf, sem):
            pltpu.make_async_copy(x_hbm.at[src_v.at[0]], buf, sem).wait()

        def scat(a, buf, sem):
            pltpu.async_copy(buf, accum_sh.at[dst_v.at[a]], sem, add=True)

        def swait(buf, sem):
            pltpu.make_async_copy(buf, accum_sh.at[dst_v.at[0]], sem).wait()

        NT = (NCHH + 1) // 2

        def body(t2, carry):
            a = 2 * t2

            @pl.when(t2 > 0)
            def _():
                swait(rows1, ssem1)

            @pl.when(a + 1 < NCHH)
            def _():
                gather(a + 1, rows1, gsem1)

            gwait(rows0, gsem0)
            scat(a, rows0, ssem0)
            swait(rows0, ssem0)

            @pl.when(a + 2 < NCHH)
            def _():
                gather(a + 2, rows0, gsem0)

            @pl.when(a + 1 < NCHH)
            def _():
                gwait(rows1, gsem1)
                scat(a + 1, rows1, ssem1)

            return carry

        for qp in range(NP):
            if qp > 0:
                pltpu.sync_copy(src_hbm.at[w, qp], src_v)
                pltpu.sync_copy(dst_hbm.at[w, qp], dst_v)
                gather(0, rows0, gsem0)
            lax.fori_loop(0, NT, body, 0)
        plsc.subcore_barrier()
        for k in range(KMAX):
            j = s + NS * k

            @pl.when(j < NB)
            def _():
                r0 = j * RB
                pltpu.sync_copy(accum_sh.at[pl.ds(r0, RB)], rows0)
                pltpu.sync_copy(rows0, out_hbm.at[c, pl.ds(r0, RB)])

    return agg_kernel(x, src_arr, dst_arr)


def _layer0_body(a_ref, x_ref, w1_ref, b1_ref, w2_ref, b2_ref, g_ref, be_ref,
                 al_ref, lw_ref, lb_ref, h_out_ref, head_ref):
    z = a_ref[0] + a_ref[1] + x_ref[...]
    t = jnp.maximum(jnp.dot(z, w1_ref[...], preferred_element_type=jnp.float32)
                    + b1_ref[...], 0.0)
    u = jnp.dot(t, w2_ref[...], preferred_element_type=jnp.float32) + b2_ref[...]
    m = jnp.mean(u, axis=0, keepdims=True)
    v = jnp.mean((u - m) ** 2, axis=0, keepdims=True)
    bn = (u - m) / jnp.sqrt(v + 1e-5) * g_ref[...] + be_ref[...]
    h_out_ref[...] = jnp.where(bn > 0, bn, al_ref[...] * bn)
    pooled = jnp.max(x_ref[...], axis=0, keepdims=True)
    head_ref[...] = (jnp.dot(pooled, lw_ref[...], preferred_element_type=jnp.float32)
                     + lb_ref[...])


def _layer1_body(a_ref, x_ref, w1_ref, b1_ref, w2_ref, b2_ref, g_ref, be_ref,
                 al_ref, lwx_ref, lbx_ref, lwh_ref, lbh_ref,
                 headx_ref, headh_ref):
    z = a_ref[0] + a_ref[1] + x_ref[...]
    t = jnp.maximum(jnp.dot(z, w1_ref[...], preferred_element_type=jnp.float32)
                    + b1_ref[...], 0.0)
    u = jnp.dot(t, w2_ref[...], preferred_element_type=jnp.float32) + b2_ref[...]
    m = jnp.mean(u, axis=0, keepdims=True)
    v = jnp.mean((u - m) ** 2, axis=0, keepdims=True)
    bn = (u - m) / jnp.sqrt(v + 1e-5) * g_ref[...] + be_ref[...]
    hn = jnp.where(bn > 0, bn, al_ref[...] * bn)
    pooledx = jnp.max(x_ref[...], axis=0, keepdims=True)
    headx_ref[...] = (jnp.dot(pooledx, lwx_ref[...],
                              preferred_element_type=jnp.float32) + lbx_ref[...])
    pooledh = jnp.max(hn, axis=0, keepdims=True)
    headh_ref[...] = (jnp.dot(pooledh, lwh_ref[...],
                              preferred_element_type=jnp.float32) + lbh_ref[...])


def kernel(h, edge_index, W1_0, b1_0, W2_0, b2_0, gamma_0, beta_0,
           W1_1, b1_1, W2_1, b2_1, gamma_1, beta_1, prelu_a,
           LW0, Lb0, LW1, Lb1, LW2, Lb2):
    alpha = jnp.broadcast_to(prelu_a, (1, D)).astype(jnp.float32)
    r = lambda v: jnp.reshape(v, (1, D))
    src_arr = edge_index[0].reshape(NW, NP, NCHH, CH)
    dst_arr = edge_index[1].reshape(NW, NP, NCHH, CH)

    a = _sc_aggregate(h, src_arr, dst_arr)
    h1, head0 = pl.pallas_call(
        _layer0_body,
        out_shape=[jax.ShapeDtypeStruct((N, D), jnp.float32),
                   jax.ShapeDtypeStruct((1, D), jnp.float32)],
    )(a, h, W1_0, r(b1_0), W2_0, r(b2_0), r(gamma_0), r(beta_0), alpha,
      LW0, r(Lb0))

    b = _sc_aggregate(h1, src_arr, dst_arr)
    head1, head2 = pl.pallas_call(
        _layer1_body,
        out_shape=[jax.ShapeDtypeStruct((1, D), jnp.float32),
                   jax.ShapeDtypeStruct((1, D), jnp.float32)],
    )(b, h1, W1_1, r(b1_1), W2_1, r(b2_1), r(gamma_1), r(beta_1), alpha,
      LW1, r(Lb1), LW2, r(Lb2))

    stacked = jnp.stack([head0, head1, head2], axis=-1)
    return stacked.reshape(1, -1)


# speedup vs baseline: 3.8969x; 1.0144x over previous
import functools

import jax
import jax.numpy as jnp
from jax import lax
from jax.experimental import pallas as pl
from jax.experimental.pallas import tpu as pltpu
from jax.experimental.pallas import tpu_sc as plsc

N = 10000
E = 320000
D = 128

NC = 2
NS = 16
NW = NC * NS

EW = E // NW
CH = 80
NP = 5
NCHH = 25
NA = N

RB = 80
NB = N // RB
KMAX = -(-NB // NS)


def _sc_aggregate(x, src_arr, dst_arr):
    mesh = plsc.VectorSubcoreMesh(core_axis_name="c", subcore_axis_name="s")

    @functools.partial(
        pl.kernel,
        out_type=jax.ShapeDtypeStruct((NC, N, D), jnp.float32),
        mesh=mesh,
        scratch_types=[
            pltpu.VMEM((NCHH, CH), jnp.int32),
            pltpu.VMEM((NCHH, CH), jnp.int32),
            pltpu.VMEM((CH, D), jnp.float32),
            pltpu.VMEM((CH, D), jnp.float32),
            pltpu.VMEM_SHARED((NA, D), jnp.float32),
            pltpu.SemaphoreType.DMA,
            pltpu.SemaphoreType.DMA,
            pltpu.SemaphoreType.DMA,
            pltpu.SemaphoreType.DMA,
        ],
    )
    def agg_kernel(x_hbm, src_hbm, dst_hbm, out_hbm, src_v, dst_v, rows0,
                   rows1, accum_sh, gsem0, gsem1, ssem0, ssem1):
        c = lax.axis_index("c")
        s = lax.axis_index("s")
        w = c * NS + s
        pltpu.sync_copy(src_hbm.at[w, 0], src_v)
        pltpu.sync_copy(dst_hbm.at[w, 0], dst_v)
        pltpu.async_copy(x_hbm.at[src_v.at[0]], rows0, gsem0)
        zv = jnp.zeros((16,), jnp.float32)

        def zbody(i, carry):
            rows1[i // 8, pl.ds((i % 8) * 16, 16)] = zv
            return carry

        lax.fori_loop(0, RB * 8, zbody, 0)
        for k in range(KMAX):
            j = s + NS * k

            @pl.when(j < NB)
            def _():
                pltpu.async_copy(rows1, accum_sh.at[pl.ds(j * RB, RB)], ssem0)

        for k in range(KMAX):
            j = s + NS * k

            @pl.when(j < NB)
            def _():
                pltpu.make_async_copy(rows1,
                                      accum_sh.at[pl.ds(0, RB)], ssem0).wait()

        plsc.subcore_barrier()

        def gather(a, buf, sem):
            pltpu.async_copy(x_hbm.at[src_v.at[a]], buf, sem)

        def gwait(buf, sem):
            pltpu.make_async_copy(x_hbm.at[src_v.at[0]], buf, sem).wait()

        def scat(a, buf, sem):
            pltpu.async_copy(buf, accum_sh.at[dst_v.at[a]], sem, add=True)

        def swait(buf, sem):
            pltpu.make_async_copy(buf, accum_sh.at[dst_v.at[0]], sem).wait()

        NT = (NCHH + 1) // 2

        def body(t2, carry):
            a = 2 * t2

            @pl.when(t2 > 0)
            def _():
                swait(rows1, ssem1)

            @pl.when(a + 1 < NCHH)
            def _():
                gather(a + 1, rows1, gsem1)

            gwait(rows0, gsem0)
            scat(a, rows0, ssem0)
            swait(rows0, ssem0)

            @pl.when(a + 2 < NCHH)
            def _():
                gather(a + 2, rows0, gsem0)

            @pl.when(a + 1 < NCHH)
            def _():
                gwait(rows1, gsem1)
                scat(a + 1, rows1, ssem1)

            return carry

        for qp in range(NP):
            if qp > 0:
                pltpu.sync_copy(src_hbm.at[w, qp], src_v)
                pltpu.sync_copy(dst_hbm.at[w, qp], dst_v)
                gather(0, rows0, gsem0)
            lax.fori_loop(0, NT, body, 0)
        plsc.subcore_barrier()
        wrows = [rows0, rows1]
        wgsem = [gsem0, gsem1]
        wssem = [ssem0, ssem1]
        for k in range(KMAX + 1):
            if k < KMAX:
                j = s + NS * k
                b = k % 2

                @pl.when(j < NB)
                def _():
                    if k >= 2:
                        pltpu.make_async_copy(
                            wrows[b], out_hbm.at[c, pl.ds(0, RB)],
                            wssem[b]).wait()
                    pltpu.async_copy(accum_sh.at[pl.ds(j * RB, RB)], wrows[b],
                                     wgsem[b])
            if k >= 1:
                kp = k - 1
                jp = s + NS * kp
                bp = kp % 2

                @pl.when(jp < NB)
                def _():
                    pltpu.make_async_copy(accum_sh.at[pl.ds(0, RB)],
                                          wrows[bp], wgsem[bp]).wait()
                    pltpu.async_copy(wrows[bp],
                                     out_hbm.at[c, pl.ds(jp * RB, RB)],
                                     wssem[bp])
        for kp in range(KMAX):
            jp = s + NS * kp
            j2 = s + NS * (kp + 2)

            @pl.when((jp < NB) & (j2 >= NB))
            def _():
                pltpu.make_async_copy(wrows[kp % 2],
                                      out_hbm.at[c, pl.ds(0, RB)],
                                      wssem[kp % 2]).wait()

    return agg_kernel(x, src_arr, dst_arr)


def _layer0_body(a_ref, x_ref, w1_ref, b1_ref, w2_ref, b2_ref, g_ref, be_ref,
                 al_ref, lw_ref, lb_ref, h_out_ref, head_ref):
    z = a_ref[0] + a_ref[1] + x_ref[...]
    t = jnp.maximum(jnp.dot(z, w1_ref[...], preferred_element_type=jnp.float32)
                    + b1_ref[...], 0.0)
    u = jnp.dot(t, w2_ref[...], preferred_element_type=jnp.float32) + b2_ref[...]
    m = jnp.mean(u, axis=0, keepdims=True)
    v = jnp.mean((u - m) ** 2, axis=0, keepdims=True)
    bn = (u - m) / jnp.sqrt(v + 1e-5) * g_ref[...] + be_ref[...]
    h_out_ref[...] = jnp.where(bn > 0, bn, al_ref[...] * bn)
    pooled = jnp.max(x_ref[...], axis=0, keepdims=True)
    head_ref[...] = (jnp.dot(pooled, lw_ref[...], preferred_element_type=jnp.float32)
                     + lb_ref[...])


def _layer1_body(a_ref, x_ref, w1_ref, b1_ref, w2_ref, b2_ref, g_ref, be_ref,
                 al_ref, lwx_ref, lbx_ref, lwh_ref, lbh_ref,
                 headx_ref, headh_ref):
    z = a_ref[0] + a_ref[1] + x_ref[...]
    t = jnp.maximum(jnp.dot(z, w1_ref[...], preferred_element_type=jnp.float32)
                    + b1_ref[...], 0.0)
    u = jnp.dot(t, w2_ref[...], preferred_element_type=jnp.float32) + b2_ref[...]
    m = jnp.mean(u, axis=0, keepdims=True)
    v = jnp.mean((u - m) ** 2, axis=0, keepdims=True)
    bn = (u - m) / jnp.sqrt(v + 1e-5) * g_ref[...] + be_ref[...]
    hn = jnp.where(bn > 0, bn, al_ref[...] * bn)
    pooledx = jnp.max(x_ref[...], axis=0, keepdims=True)
    headx_ref[...] = (jnp.dot(pooledx, lwx_ref[...],
                              preferred_element_type=jnp.float32) + lbx_ref[...])
    pooledh = jnp.max(hn, axis=0, keepdims=True)
    headh_ref[...] = (jnp.dot(pooledh, lwh_ref[...],
                              preferred_element_type=jnp.float32) + lbh_ref[...])


def kernel(h, edge_index, W1_0, b1_0, W2_0, b2_0, gamma_0, beta_0,
           W1_1, b1_1, W2_1, b2_1, gamma_1, beta_1, prelu_a,
           LW0, Lb0, LW1, Lb1, LW2, Lb2):
    alpha = jnp.broadcast_to(prelu_a, (1, D)).astype(jnp.float32)
    r = lambda v: jnp.reshape(v, (1, D))
    src_arr = edge_index[0].reshape(NW, NP, NCHH, CH)
    dst_arr = edge_index[1].reshape(NW, NP, NCHH, CH)

    a = _sc_aggregate(h, src_arr, dst_arr)
    h1, head0 = pl.pallas_call(
        _layer0_body,
        out_shape=[jax.ShapeDtypeStruct((N, D), jnp.float32),
                   jax.ShapeDtypeStruct((1, D), jnp.float32)],
    )(a, h, W1_0, r(b1_0), W2_0, r(b2_0), r(gamma_0), r(beta_0), alpha,
      LW0, r(Lb0))

    b = _sc_aggregate(h1, src_arr, dst_arr)
    head1, head2 = pl.pallas_call(
        _layer1_body,
        out_shape=[jax.ShapeDtypeStruct((1, D), jnp.float32),
                   jax.ShapeDtypeStruct((1, D), jnp.float32)],
    )(b, h1, W1_1, r(b1_1), W2_1, r(b2_1), r(gamma_1), r(beta_1), alpha,
      LW1, r(Lb1), LW2, r(Lb2))

    stacked = jnp.stack([head0, head1, head2], axis=-1)
    return stacked.reshape(1, -1)
